# baseline (speedup 1.0000x reference)
import functools

import jax
import jax.numpy as jnp
from jax import lax
from jax.experimental import pallas as pl
from jax.experimental.pallas import tpu as pltpu
from jax.experimental.pallas import tpu_sc as plsc

N = 10000
E = 320000
D = 128
DH = D // 2
NPAD = 10240

NC = 2
NS = 16
NW = NC * NS
RB = NPAD // NS

EC = 125
EROWS = E // EC
CPT_DEG = EROWS // NW
CPT = EROWS // NS
NBUF = 5
GAH = 2
NOUT = CPT // NBUF

TCB = 1024
GRID = NPAD // TCB
DEGB = TCB // 128


def _sc_degree_body(dst_hbm, ones_hbm, zeros_hbm, deg_hbm, idx_v, ones_v,
                    deg_sh, sem):
    cid = lax.axis_index("c")
    sid = lax.axis_index("s")
    wid = sid * NC + cid

    pltpu.sync_copy(zeros_hbm.at[pl.ds(sid * RB, RB)],
                    deg_sh.at[pl.ds(sid * RB, RB)])
    pltpu.sync_copy(ones_hbm, ones_v)
    plsc.subcore_barrier()
    pltpu.sync_copy(dst_hbm.at[pl.ds(wid * CPT_DEG, CPT_DEG)], idx_v)

    def chunk(j, carry):
        pltpu.sync_copy(ones_v, deg_sh.at[idx_v.at[j]], add=True)
        return carry

    lax.fori_loop(0, CPT_DEG, chunk, 0)
    plsc.subcore_barrier()
    pltpu.sync_copy(deg_sh.at[pl.ds(sid * RB, RB)],
                    deg_hbm.at[pl.ds(cid * NPAD + sid * RB, RB)])


def _sc_agg_body(ysa_hbm, ysb_hbm, src_hbm, dst_hbm, zeros_hbm, p_hbm,
                 src_v, dst_v, r0, r1, r2, r3, r4, acc_sh,
                 g0, g1, g2, g3, g4, s0, s1, s2, s3, s4):
    rows = (r0, r1, r2, r3, r4)
    gsem = (g0, g1, g2, g3, g4)
    ssem = (s0, s1, s2, s3, s4)
    cid = lax.axis_index("c")
    sid = lax.axis_index("s")

    pltpu.sync_copy(zeros_hbm, acc_sh.at[pl.ds(sid * RB, RB)])
    base = sid * CPT
    pltpu.sync_copy(src_hbm.at[pl.ds(base, CPT)], src_v)
    pltpu.sync_copy(dst_hbm.at[pl.ds(base, CPT)], dst_v)
    plsc.subcore_barrier()

    def ring(ys_hbm):
        for c in range(GAH):
            pltpu.async_copy(ys_hbm.at[src_v.at[c]], rows[c], gsem[c])

        def outer(i, carry):
            for b in range(NBUF):
                j = i * NBUF + b
                nb = (b + GAH) % NBUF

                if b < NBUF - GAH:
                    @pl.when(i > 0)
                    def _wait_old():
                        pltpu.make_async_copy(
                            rows[nb], acc_sh.at[dst_v.at[j - (NBUF - GAH)]],
                            ssem[nb]).wait()
                else:
                    pltpu.make_async_copy(
                        rows[nb], acc_sh.at[dst_v.at[j - (NBUF - GAH)]],
                        ssem[nb]).wait()

                if b < NBUF - GAH:
                    pltpu.async_copy(ys_hbm.at[src_v.at[j + GAH]], rows[nb],
                                     gsem[nb])
                else:
                    @pl.when(i < NOUT - 1)
                    def _prefetch():
                        pltpu.async_copy(ys_hbm.at[src_v.at[j + GAH]],
                                        rows[nb], gsem[nb])

                pltpu.make_async_copy(ys_hbm.at[src_v.at[j]], rows[b],
                                      gsem[b]).wait()
                pltpu.async_copy(rows[b], acc_sh.at[dst_v.at[j]], ssem[b],
                                 add=True)
            return carry

        lax.fori_loop(0, NOUT, outer, 0)

        for k in range(NBUF - GAH):
            c = CPT - (NBUF - GAH) + k
            pltpu.make_async_copy(rows[c % NBUF], acc_sh.at[dst_v.at[c]],
                                  ssem[c % NBUF]).wait()

    @pl.when(cid == 0)
    def _core0():
        ring(ysa_hbm)

    @pl.when(cid == 1)
    def _core1():
        ring(ysb_hbm)

    plsc.subcore_barrier()
    pltpu.sync_copy(acc_sh.at[pl.ds(sid * RB, RB)],
                    p_hbm.at[cid, pl.ds(sid * RB, RB)])


@functools.cache
def _build_sc_kernels():
    mesh = plsc.VectorSubcoreMesh(
        core_axis_name="c", subcore_axis_name="s",
        num_cores=NC, num_subcores=NS)
    sc_degree = pl.kernel(
        _sc_degree_body,
        out_type=jax.ShapeDtypeStruct((NC * NPAD,), jnp.float32),
        mesh=mesh,
        compiler_params=pltpu.CompilerParams(use_tc_tiling_on_sc=False),
        scratch_types=[
            pltpu.VMEM((CPT_DEG, EC), jnp.int32),
            pltpu.VMEM((EC,), jnp.float32),
            pltpu.VMEM_SHARED((NPAD,), jnp.float32),
            pltpu.SemaphoreType.DMA,
        ],
    )
    sc_agg = pl.kernel(
        _sc_agg_body,
        out_type=jax.ShapeDtypeStruct((NC, NPAD, DH), jnp.float32),
        mesh=mesh,
        compiler_params=pltpu.CompilerParams(use_tc_tiling_on_sc=False),
        scratch_types=(
            [pltpu.VMEM((CPT, EC), jnp.int32),
             pltpu.VMEM((CPT, EC), jnp.int32)]
            + [pltpu.VMEM((EC, DH), jnp.float32) for _ in range(NBUF)]
            + [pltpu.VMEM_SHARED((NPAD, DH), jnp.float32)]
            + [pltpu.SemaphoreType.DMA for _ in range(2 * NBUF)]
        ),
    )
    return sc_degree, sc_agg


def _tc_l1_body(x_ref, w_ref, dg_ref, ysa_ref, ysb_ref):
    dis = lax.rsqrt(dg_ref[...] + 1.0)
    y = jnp.dot(x_ref[...], w_ref[...], preferred_element_type=jnp.float32)
    ys = y * dis
    ysa_ref[...] = ys[:, :DH]
    ysb_ref[...] = ys[:, DH:]


def _tc_l2_body(pa_ref, pb_ref, ysa_ref, ysb_ref, dg_ref, b1_ref,
                w2_ref, y2sa_ref, y2sb_ref):
    dis = lax.rsqrt(dg_ref[...] + 1.0)
    p = jnp.concatenate([pa_ref[0], pb_ref[0]], axis=1)
    ys = jnp.concatenate([ysa_ref[...], ysb_ref[...]], axis=1)
    h = dis * (p + ys) + b1_ref[...]
    h = jnp.maximum(h, 0.0)
    y2s = jnp.dot(h, w2_ref[...], preferred_element_type=jnp.float32) * dis
    y2sa_ref[...] = y2s[:, :DH]
    y2sb_ref[...] = y2s[:, DH:]


def _tc_out_body(pa_ref, pb_ref, y2sa_ref, y2sb_ref, dg_ref, b2_ref,
                 out_ref):
    dis = lax.rsqrt(dg_ref[...] + 1.0)
    p = jnp.concatenate([pa_ref[0], pb_ref[0]], axis=1)
    y2s = jnp.concatenate([y2sa_ref[...], y2sb_ref[...]], axis=1)
    out_ref[...] = dis * (p + y2s) + b2_ref[...]


_row_spec = pl.BlockSpec((TCB, D), lambda i: (i, 0))
_half_spec = pl.BlockSpec((TCB, DH), lambda i: (i, 0))
_pa_spec = pl.BlockSpec((1, TCB, DH), lambda i: (0, i, 0))
_pb_spec = pl.BlockSpec((1, TCB, DH), lambda i: (1, i, 0))
_dg_spec = pl.BlockSpec((TCB, D), lambda i: (i, 0))
_w_spec = pl.BlockSpec((D, D), lambda i: (0, 0))
_b_spec = pl.BlockSpec((1, D), lambda i: (0, 0))

_tc_l1 = pl.pallas_call(
    _tc_l1_body,
    grid=(GRID,),
    in_specs=[_row_spec, _w_spec, _dg_spec],
    out_specs=[_half_spec, _half_spec],
    out_shape=[jax.ShapeDtypeStruct((NPAD, DH), jnp.float32),
               jax.ShapeDtypeStruct((NPAD, DH), jnp.float32)],
)

_tc_l2 = pl.pallas_call(
    _tc_l2_body,
    grid=(GRID,),
    in_specs=[_pa_spec, _pb_spec, _half_spec, _half_spec, _dg_spec,
              _b_spec, _w_spec],
    out_specs=[_half_spec, _half_spec],
    out_shape=[jax.ShapeDtypeStruct((NPAD, DH), jnp.float32),
               jax.ShapeDtypeStruct((NPAD, DH), jnp.float32)],
)

_tc_out = pl.pallas_call(
    _tc_out_body,
    grid=(GRID,),
    in_specs=[_pa_spec, _pb_spec, _half_spec, _half_spec, _dg_spec,
              _b_spec],
    out_specs=_row_spec,
    out_shape=jax.ShapeDtypeStruct((NPAD, D), jnp.float32),
)


def kernel(x, edge_index, W1, b1, W2, b2):
    x_pad = jnp.zeros((NPAD, D), jnp.float32).at[:N].set(x)
    src2 = edge_index[0].reshape(EROWS, EC)
    dst2 = edge_index[1].reshape(EROWS, EC)
    zeros2 = jnp.zeros((RB, DH), jnp.float32)
    zeros1 = jnp.zeros((NPAD,), jnp.float32)
    ones_c = jnp.ones((EC,), jnp.float32)
    b1r = b1.reshape(1, D)
    b2r = b2.reshape(1, D)

    _sc_degree, _sc_agg = _build_sc_kernels()
    deg = _sc_degree(dst2, ones_c, zeros1)
    deg_bc = jnp.broadcast_to((deg[:NPAD] + deg[NPAD:])[:, None], (NPAD, D))

    y1sa, y1sb = _tc_l1(x_pad, W1, deg_bc)
    p1 = _sc_agg(y1sa, y1sb, src2, dst2, zeros2)
    y2sa, y2sb = _tc_l2(p1, p1, y1sa, y1sb, deg_bc, b1r, W2)
    p2 = _sc_agg(y2sa, y2sb, src2, dst2, zeros2)
    out = _tc_out(p2, p2, y2sa, y2sb, deg_bc, b2r)
    return out[:N]


# speedup vs baseline: 1.0436x; 1.0436x over previous
import functools

import jax
import jax.numpy as jnp
from jax import lax
from jax.experimental import pallas as pl
from jax.experimental.pallas import tpu as pltpu
from jax.experimental.pallas import tpu_sc as plsc

N = 10000
E = 320000
D = 128
DH = D // 2
NPAD = 10240

NC = 2
NS = 16
NW = NC * NS
RB = NPAD // NS

EC = 125
EROWS = E // EC
CPT_DEG = EROWS // NW
CPT = EROWS // NS
NBUF = 5
GAH = 3
NOUT = CPT // NBUF

TCB = 1024
GRID = NPAD // TCB
DEGB = TCB // 128


def _sc_degree_body(edges_hbm, ones_hbm, zeros_hbm, deg_hbm, idx_v, ones_v,
                    deg_sh, sem):
    cid = lax.axis_index("c")
    sid = lax.axis_index("s")
    wid = sid * NC + cid

    pltpu.sync_copy(zeros_hbm.at[pl.ds(sid * RB, RB)],
                    deg_sh.at[pl.ds(sid * RB, RB)])
    pltpu.sync_copy(ones_hbm, ones_v)
    plsc.subcore_barrier()
    pltpu.sync_copy(edges_hbm.at[pl.ds(EROWS + wid * CPT_DEG, CPT_DEG)],
                    idx_v)

    def chunk(j, carry):
        pltpu.sync_copy(ones_v, deg_sh.at[idx_v.at[j]], add=True)
        return carry

    lax.fori_loop(0, CPT_DEG, chunk, 0)
    plsc.subcore_barrier()
    pltpu.sync_copy(deg_sh.at[pl.ds(sid * RB, RB)],
                    deg_hbm.at[pl.ds(cid * NPAD + sid * RB, RB)])


def _sc_agg_body(ysa_hbm, ysb_hbm, edges_hbm, zeros_hbm, p_hbm,
                 src_v, dst_v, r0, r1, r2, r3, r4, acc_sh,
                 g0, g1, g2, g3, g4, s0, s1, s2, s3, s4):
    rows = (r0, r1, r2, r3, r4)
    gsem = (g0, g1, g2, g3, g4)
    ssem = (s0, s1, s2, s3, s4)
    cid = lax.axis_index("c")
    sid = lax.axis_index("s")

    pltpu.sync_copy(zeros_hbm, acc_sh.at[pl.ds(sid * RB, RB)])
    base = sid * CPT
    pltpu.sync_copy(edges_hbm.at[pl.ds(base, CPT)], src_v)
    pltpu.sync_copy(edges_hbm.at[pl.ds(EROWS + base, CPT)], dst_v)
    plsc.subcore_barrier()

    def ring(ys_hbm):
        for c in range(GAH):
            pltpu.async_copy(ys_hbm.at[src_v.at[c]], rows[c], gsem[c])

        def outer(i, carry):
            for b in range(NBUF):
                j = i * NBUF + b
                nb = (b + GAH) % NBUF

                if b < NBUF - GAH:
                    @pl.when(i > 0)
                    def _wait_old():
                        pltpu.make_async_copy(
                            rows[nb], acc_sh.at[dst_v.at[j - (NBUF - GAH)]],
                            ssem[nb]).wait()
                else:
                    pltpu.make_async_copy(
                        rows[nb], acc_sh.at[dst_v.at[j - (NBUF - GAH)]],
                        ssem[nb]).wait()

                if b < NBUF - GAH:
                    pltpu.async_copy(ys_hbm.at[src_v.at[j + GAH]], rows[nb],
                                     gsem[nb])
                else:
                    @pl.when(i < NOUT - 1)
                    def _prefetch():
                        pltpu.async_copy(ys_hbm.at[src_v.at[j + GAH]],
                                        rows[nb], gsem[nb])

                pltpu.make_async_copy(ys_hbm.at[src_v.at[j]], rows[b],
                                      gsem[b]).wait()
                pltpu.async_copy(rows[b], acc_sh.at[dst_v.at[j]], ssem[b],
                                 add=True)
            return carry

        lax.fori_loop(0, NOUT, outer, 0)

        for k in range(NBUF - GAH):
            c = CPT - (NBUF - GAH) + k
            pltpu.make_async_copy(rows[c % NBUF], acc_sh.at[dst_v.at[c]],
                                  ssem[c % NBUF]).wait()

    @pl.when(cid == 0)
    def _core0():
        ring(ysa_hbm)

    @pl.when(cid == 1)
    def _core1():
        ring(ysb_hbm)

    plsc.subcore_barrier()
    pltpu.sync_copy(acc_sh.at[pl.ds(sid * RB, RB)],
                    p_hbm.at[cid, pl.ds(sid * RB, RB)])


@functools.cache
def _build_sc_kernels():
    mesh = plsc.VectorSubcoreMesh(
        core_axis_name="c", subcore_axis_name="s",
        num_cores=NC, num_subcores=NS)
    sc_degree = pl.kernel(
        _sc_degree_body,
        out_type=jax.ShapeDtypeStruct((NC * NPAD,), jnp.float32),
        mesh=mesh,
        compiler_params=pltpu.CompilerParams(use_tc_tiling_on_sc=False),
        scratch_types=[
            pltpu.VMEM((CPT_DEG, EC), jnp.int32),
            pltpu.VMEM((EC,), jnp.float32),
            pltpu.VMEM_SHARED((NPAD,), jnp.float32),
            pltpu.SemaphoreType.DMA,
        ],
    )
    sc_agg = pl.kernel(
        _sc_agg_body,
        out_type=jax.ShapeDtypeStruct((NC, NPAD, DH), jnp.float32),
        mesh=mesh,
        compiler_params=pltpu.CompilerParams(use_tc_tiling_on_sc=False),
        scratch_types=(
            [pltpu.VMEM((CPT, EC), jnp.int32),
             pltpu.VMEM((CPT, EC), jnp.int32)]
            + [pltpu.VMEM((EC, DH), jnp.float32) for _ in range(NBUF)]
            + [pltpu.VMEM_SHARED((NPAD, DH), jnp.float32)]
            + [pltpu.SemaphoreType.DMA for _ in range(2 * NBUF)]
        ),
    )
    return sc_degree, sc_agg


def _tc_l1_body(x_ref, w_ref, dg_ref, ysa_ref, ysb_ref):
    dis = lax.rsqrt(dg_ref[...] + 1.0)
    y = jnp.dot(x_ref[...], w_ref[...], preferred_element_type=jnp.float32)
    ys = y * dis
    ysa_ref[...] = ys[:, :DH]
    ysb_ref[...] = ys[:, DH:]


def _tc_l2_body(pa_ref, pb_ref, ysa_ref, ysb_ref, dg_ref, b1_ref,
                w2_ref, y2sa_ref, y2sb_ref):
    dis = lax.rsqrt(dg_ref[...] + 1.0)
    p = jnp.concatenate([pa_ref[0], pb_ref[0]], axis=1)
    ys = jnp.concatenate([ysa_ref[...], ysb_ref[...]], axis=1)
    h = dis * (p + ys) + b1_ref[...]
    h = jnp.maximum(h, 0.0)
    y2s = jnp.dot(h, w2_ref[...], preferred_element_type=jnp.float32) * dis
    y2sa_ref[...] = y2s[:, :DH]
    y2sb_ref[...] = y2s[:, DH:]


def _tc_out_body(pa_ref, pb_ref, y2sa_ref, y2sb_ref, dg_ref, b2_ref,
                 out_ref):
    dis = lax.rsqrt(dg_ref[...] + 1.0)
    p = jnp.concatenate([pa_ref[0], pb_ref[0]], axis=1)
    y2s = jnp.concatenate([y2sa_ref[...], y2sb_ref[...]], axis=1)
    out_ref[...] = dis * (p + y2s) + b2_ref[...]


_row_spec = pl.BlockSpec((TCB, D), lambda i: (i, 0))
_half_spec = pl.BlockSpec((TCB, DH), lambda i: (i, 0))
_pa_spec = pl.BlockSpec((1, TCB, DH), lambda i: (0, i, 0))
_pb_spec = pl.BlockSpec((1, TCB, DH), lambda i: (1, i, 0))
_dg_spec = pl.BlockSpec((TCB, D), lambda i: (i, 0))
_w_spec = pl.BlockSpec((D, D), lambda i: (0, 0))
_b_spec = pl.BlockSpec((1, D), lambda i: (0, 0))

_tc_l1 = pl.pallas_call(
    _tc_l1_body,
    grid=(GRID,),
    in_specs=[_row_spec, _w_spec, _dg_spec],
    out_specs=[_half_spec, _half_spec],
    out_shape=[jax.ShapeDtypeStruct((NPAD, DH), jnp.float32),
               jax.ShapeDtypeStruct((NPAD, DH), jnp.float32)],
)

_tc_l2 = pl.pallas_call(
    _tc_l2_body,
    grid=(GRID,),
    in_specs=[_pa_spec, _pb_spec, _half_spec, _half_spec, _dg_spec,
              _b_spec, _w_spec],
    out_specs=[_half_spec, _half_spec],
    out_shape=[jax.ShapeDtypeStruct((NPAD, DH), jnp.float32),
               jax.ShapeDtypeStruct((NPAD, DH), jnp.float32)],
)

_tc_out = pl.pallas_call(
    _tc_out_body,
    grid=(GRID,),
    in_specs=[_pa_spec, _pb_spec, _half_spec, _half_spec, _dg_spec,
              _b_spec],
    out_specs=_row_spec,
    out_shape=jax.ShapeDtypeStruct((NPAD, D), jnp.float32),
)


def kernel(x, edge_index, W1, b1, W2, b2):
    x_pad = jnp.zeros((NPAD, D), jnp.float32).at[:N].set(x)
    edges2 = edge_index.reshape(2 * EROWS, EC)
    zeros2 = jnp.zeros((RB, DH), jnp.float32)
    zeros1 = jnp.zeros((NPAD,), jnp.float32)
    ones_c = jnp.ones((EC,), jnp.float32)
    b1r = b1.reshape(1, D)
    b2r = b2.reshape(1, D)

    _sc_degree, _sc_agg = _build_sc_kernels()
    deg = _sc_degree(edges2, ones_c, zeros1)
    deg_bc = jnp.broadcast_to((deg[:NPAD] + deg[NPAD:])[:, None], (NPAD, D))

    y1sa, y1sb = _tc_l1(x_pad, W1, deg_bc)
    p1 = _sc_agg(y1sa, y1sb, edges2, zeros2)
    y2sa, y2sb = _tc_l2(p1, p1, y1sa, y1sb, deg_bc, b1r, W2)
    p2 = _sc_agg(y2sa, y2sb, edges2, zeros2)
    out = _tc_out(p2, p2, y2sa, y2sb, deg_bc, b2r)
    return out[:N]


# speedup vs baseline: 1.0658x; 1.0212x over previous
import functools

import jax
import jax.numpy as jnp
from jax import lax
from jax.experimental import pallas as pl
from jax.experimental.pallas import tpu as pltpu
from jax.experimental.pallas import tpu_sc as plsc

N = 10000
E = 320000
D = 128
DH = D // 2
NPAD = 10240

NC = 2
NS = 16
NW = NC * NS
RB = NPAD // NS
NSUB = N // NS

EC = 125
EROWS = E // EC
CPT_DEG = EROWS // NW
CPT = EROWS // NS
NBUF = 5
GAH = 3
NOUT = CPT // NBUF

TCB = 1000
GRID = N // TCB
DEGB = TCB // 128


def _sc_degree_body(edges_hbm, ones_hbm, zeros_hbm, deg_hbm, idx_v, ones_v,
                    deg_sh, sem):
    cid = lax.axis_index("c")
    sid = lax.axis_index("s")
    wid = sid * NC + cid

    pltpu.sync_copy(zeros_hbm.at[pl.ds(sid * RB, RB)],
                    deg_sh.at[pl.ds(sid * RB, RB)])
    pltpu.sync_copy(ones_hbm, ones_v)
    plsc.subcore_barrier()
    pltpu.sync_copy(edges_hbm.at[pl.ds(EROWS + wid * CPT_DEG, CPT_DEG)],
                    idx_v)

    def chunk(j, carry):
        pltpu.sync_copy(ones_v, deg_sh.at[idx_v.at[j]], add=True)
        return carry

    lax.fori_loop(0, CPT_DEG, chunk, 0)
    plsc.subcore_barrier()
    pltpu.sync_copy(deg_sh.at[pl.ds(sid * RB, RB)],
                    deg_hbm.at[pl.ds(cid * NPAD + sid * RB, RB)])


def _sc_agg_body(ysa_hbm, ysb_hbm, edges_hbm, zeros_hbm, p_hbm,
                 src_v, dst_v, r0, r1, r2, r3, r4, acc_sh,
                 g0, g1, g2, g3, g4, s0, s1, s2, s3, s4):
    rows = (r0, r1, r2, r3, r4)
    gsem = (g0, g1, g2, g3, g4)
    ssem = (s0, s1, s2, s3, s4)
    cid = lax.axis_index("c")
    sid = lax.axis_index("s")

    pltpu.sync_copy(zeros_hbm, acc_sh.at[pl.ds(sid * NSUB, NSUB)])
    base = sid * CPT
    pltpu.sync_copy(edges_hbm.at[pl.ds(base, CPT)], src_v)
    pltpu.sync_copy(edges_hbm.at[pl.ds(EROWS + base, CPT)], dst_v)
    plsc.subcore_barrier()

    def ring(ys_hbm):
        for c in range(GAH):
            pltpu.async_copy(ys_hbm.at[src_v.at[c]], rows[c], gsem[c])

        def outer(i, carry):
            for b in range(NBUF):
                j = i * NBUF + b
                nb = (b + GAH) % NBUF

                if b < NBUF - GAH:
                    @pl.when(i > 0)
                    def _wait_old():
                        pltpu.make_async_copy(
                            rows[nb], acc_sh.at[dst_v.at[j - (NBUF - GAH)]],
                            ssem[nb]).wait()
                else:
                    pltpu.make_async_copy(
                        rows[nb], acc_sh.at[dst_v.at[j - (NBUF - GAH)]],
                        ssem[nb]).wait()

                if b < NBUF - GAH:
                    pltpu.async_copy(ys_hbm.at[src_v.at[j + GAH]], rows[nb],
                                     gsem[nb])
                else:
                    @pl.when(i < NOUT - 1)
                    def _prefetch():
                        pltpu.async_copy(ys_hbm.at[src_v.at[j + GAH]],
                                        rows[nb], gsem[nb])

                pltpu.make_async_copy(ys_hbm.at[src_v.at[j]], rows[b],
                                      gsem[b]).wait()
                pltpu.async_copy(rows[b], acc_sh.at[dst_v.at[j]], ssem[b],
                                 add=True)
            return carry

        lax.fori_loop(0, NOUT, outer, 0)

        for k in range(NBUF - GAH):
            c = CPT - (NBUF - GAH) + k
            pltpu.make_async_copy(rows[c % NBUF], acc_sh.at[dst_v.at[c]],
                                  ssem[c % NBUF]).wait()

    @pl.when(cid == 0)
    def _core0():
        ring(ysa_hbm)

    @pl.when(cid == 1)
    def _core1():
        ring(ysb_hbm)

    plsc.subcore_barrier()
    pltpu.sync_copy(acc_sh.at[pl.ds(sid * NSUB, NSUB)],
                    p_hbm.at[cid, pl.ds(sid * NSUB, NSUB)])


@functools.cache
def _build_sc_kernels():
    mesh = plsc.VectorSubcoreMesh(
        core_axis_name="c", subcore_axis_name="s",
        num_cores=NC, num_subcores=NS)
    sc_degree = pl.kernel(
        _sc_degree_body,
        out_type=jax.ShapeDtypeStruct((NC * NPAD,), jnp.float32),
        mesh=mesh,
        compiler_params=pltpu.CompilerParams(use_tc_tiling_on_sc=False),
        scratch_types=[
            pltpu.VMEM((CPT_DEG, EC), jnp.int32),
            pltpu.VMEM((EC,), jnp.float32),
            pltpu.VMEM_SHARED((NPAD,), jnp.float32),
            pltpu.SemaphoreType.DMA,
        ],
    )
    sc_agg = pl.kernel(
        _sc_agg_body,
        out_type=jax.ShapeDtypeStruct((NC, N, DH), jnp.float32),
        mesh=mesh,
        compiler_params=pltpu.CompilerParams(use_tc_tiling_on_sc=False),
        scratch_types=(
            [pltpu.VMEM((CPT, EC), jnp.int32),
             pltpu.VMEM((CPT, EC), jnp.int32)]
            + [pltpu.VMEM((EC, DH), jnp.float32) for _ in range(NBUF)]
            + [pltpu.VMEM_SHARED((N, DH), jnp.float32)]
            + [pltpu.SemaphoreType.DMA for _ in range(2 * NBUF)]
        ),
    )
    return sc_degree, sc_agg


def _tc_l1_body(x_ref, w_ref, dg_ref, ysa_ref, ysb_ref):
    dis = lax.rsqrt(dg_ref[...] + 1.0)
    y = jnp.dot(x_ref[...], w_ref[...], preferred_element_type=jnp.float32)
    ys = y * dis
    ysa_ref[...] = ys[:, :DH]
    ysb_ref[...] = ys[:, DH:]


def _tc_l2_body(pa_ref, pb_ref, ysa_ref, ysb_ref, dg_ref, b1_ref,
                w2_ref, y2sa_ref, y2sb_ref):
    dis = lax.rsqrt(dg_ref[...] + 1.0)
    p = jnp.concatenate([pa_ref[0], pb_ref[0]], axis=1)
    ys = jnp.concatenate([ysa_ref[...], ysb_ref[...]], axis=1)
    h = dis * (p + ys) + b1_ref[...]
    h = jnp.maximum(h, 0.0)
    y2s = jnp.dot(h, w2_ref[...], preferred_element_type=jnp.float32) * dis
    y2sa_ref[...] = y2s[:, :DH]
    y2sb_ref[...] = y2s[:, DH:]


def _tc_out_body(pa_ref, pb_ref, y2sa_ref, y2sb_ref, dg_ref, b2_ref,
                 out_ref):
    dis = lax.rsqrt(dg_ref[...] + 1.0)
    p = jnp.concatenate([pa_ref[0], pb_ref[0]], axis=1)
    y2s = jnp.concatenate([y2sa_ref[...], y2sb_ref[...]], axis=1)
    out_ref[...] = dis * (p + y2s) + b2_ref[...]


_row_spec = pl.BlockSpec((TCB, D), lambda i: (i, 0))
_half_spec = pl.BlockSpec((TCB, DH), lambda i: (i, 0))
_pa_spec = pl.BlockSpec((1, TCB, DH), lambda i: (0, i, 0))
_pb_spec = pl.BlockSpec((1, TCB, DH), lambda i: (1, i, 0))
_dg_spec = pl.BlockSpec((TCB, D), lambda i: (i, 0))
_w_spec = pl.BlockSpec((D, D), lambda i: (0, 0))
_b_spec = pl.BlockSpec((1, D), lambda i: (0, 0))

_tc_l1 = pl.pallas_call(
    _tc_l1_body,
    grid=(GRID,),
    in_specs=[_row_spec, _w_spec, _dg_spec],
    out_specs=[_half_spec, _half_spec],
    out_shape=[jax.ShapeDtypeStruct((N, DH), jnp.float32),
               jax.ShapeDtypeStruct((N, DH), jnp.float32)],
)

_tc_l2 = pl.pallas_call(
    _tc_l2_body,
    grid=(GRID,),
    in_specs=[_pa_spec, _pb_spec, _half_spec, _half_spec, _dg_spec,
              _b_spec, _w_spec],
    out_specs=[_half_spec, _half_spec],
    out_shape=[jax.ShapeDtypeStruct((N, DH), jnp.float32),
               jax.ShapeDtypeStruct((N, DH), jnp.float32)],
)

_tc_out = pl.pallas_call(
    _tc_out_body,
    grid=(GRID,),
    in_specs=[_pa_spec, _pb_spec, _half_spec, _half_spec, _dg_spec,
              _b_spec],
    out_specs=_row_spec,
    out_shape=jax.ShapeDtypeStruct((N, D), jnp.float32),
)


def kernel(x, edge_index, W1, b1, W2, b2):
    edges2 = edge_index.reshape(2 * EROWS, EC)
    zeros2 = jnp.zeros((NSUB, DH), jnp.float32)
    zeros1 = jnp.zeros((NPAD,), jnp.float32)
    ones_c = jnp.ones((EC,), jnp.float32)
    b1r = b1.reshape(1, D)
    b2r = b2.reshape(1, D)

    _sc_degree, _sc_agg = _build_sc_kernels()
    deg = _sc_degree(edges2, ones_c, zeros1)
    deg_bc = jnp.broadcast_to(
        (deg[:N] + deg[NPAD:NPAD + N])[:, None], (N, D))

    y1sa, y1sb = _tc_l1(x, W1, deg_bc)
    p1 = _sc_agg(y1sa, y1sb, edges2, zeros2)
    y2sa, y2sb = _tc_l2(p1, p1, y1sa, y1sb, deg_bc, b1r, W2)
    p2 = _sc_agg(y2sa, y2sb, edges2, zeros2)
    return _tc_out(p2, p2, y2sa, y2sb, deg_bc, b2r)


# speedup vs baseline: 1.0662x; 1.0004x over previous
import functools

import jax
import jax.numpy as jnp
from jax import lax
from jax.experimental import pallas as pl
from jax.experimental.pallas import tpu as pltpu
from jax.experimental.pallas import tpu_sc as plsc

N = 10000
E = 320000
D = 128
DH = D // 2
NPAD = 10240

NC = 2
NS = 16
NW = NC * NS
RB = NPAD // NS
NSUB = N // NS

EC = 125
EROWS = E // EC
CPT_DEG = EROWS // NW
CPT = EROWS // NS
NBUF = 5
GAH = 3
NOUT = CPT // NBUF

TCB = 1000
GRID = N // TCB


def _sc_degree_body(edges_hbm, ones_hbm, zeros_hbm, deg_hbm, idx_v, ones_v,
                    deg_sh, sem):
    cid = lax.axis_index("c")
    sid = lax.axis_index("s")
    wid = sid * NC + cid

    pltpu.sync_copy(zeros_hbm.at[pl.ds(sid * RB, RB)],
                    deg_sh.at[pl.ds(sid * RB, RB)])
    pltpu.sync_copy(ones_hbm, ones_v)
    plsc.subcore_barrier()
    pltpu.sync_copy(edges_hbm.at[pl.ds(EROWS + wid * CPT_DEG, CPT_DEG)],
                    idx_v)

    def chunk(j, carry):
        pltpu.sync_copy(ones_v, deg_sh.at[idx_v.at[j]], add=True)
        return carry

    lax.fori_loop(0, CPT_DEG, chunk, 0)
    plsc.subcore_barrier()
    pltpu.sync_copy(deg_sh.at[pl.ds(sid * RB, RB)],
                    deg_hbm.at[pl.ds(cid * NPAD + sid * RB, RB)])


def _sc_agg_body(ysa_hbm, ysb_hbm, edges_hbm, zeros_hbm, p_hbm,
                 src_v, dst_v, r0, r1, r2, r3, r4, acc_sh,
                 g0, g1, g2, g3, g4, s0, s1, s2, s3, s4):
    rows = (r0, r1, r2, r3, r4)
    gsem = (g0, g1, g2, g3, g4)
    ssem = (s0, s1, s2, s3, s4)
    cid = lax.axis_index("c")
    sid = lax.axis_index("s")

    pltpu.sync_copy(zeros_hbm, acc_sh.at[pl.ds(sid * NSUB, NSUB)])
    base = sid * CPT
    pltpu.sync_copy(edges_hbm.at[pl.ds(base, CPT)], src_v)
    pltpu.sync_copy(edges_hbm.at[pl.ds(EROWS + base, CPT)], dst_v)
    plsc.subcore_barrier()

    def ring(ys_hbm):
        for c in range(GAH):
            pltpu.async_copy(ys_hbm.at[src_v.at[c]], rows[c], gsem[c])

        def outer(i, carry):
            for b in range(NBUF):
                j = i * NBUF + b
                nb = (b + GAH) % NBUF

                if b < NBUF - GAH:
                    @pl.when(i > 0)
                    def _wait_old():
                        pltpu.make_async_copy(
                            rows[nb], acc_sh.at[dst_v.at[j - (NBUF - GAH)]],
                            ssem[nb]).wait()
                else:
                    pltpu.make_async_copy(
                        rows[nb], acc_sh.at[dst_v.at[j - (NBUF - GAH)]],
                        ssem[nb]).wait()

                if b < NBUF - GAH:
                    pltpu.async_copy(ys_hbm.at[src_v.at[j + GAH]], rows[nb],
                                     gsem[nb])
                else:
                    @pl.when(i < NOUT - 1)
                    def _prefetch():
                        pltpu.async_copy(ys_hbm.at[src_v.at[j + GAH]],
                                        rows[nb], gsem[nb])

                pltpu.make_async_copy(ys_hbm.at[src_v.at[j]], rows[b],
                                      gsem[b]).wait()
                pltpu.async_copy(rows[b], acc_sh.at[dst_v.at[j]], ssem[b],
                                 add=True)
            return carry

        lax.fori_loop(0, NOUT, outer, 0)

        for k in range(NBUF - GAH):
            c = CPT - (NBUF - GAH) + k
            pltpu.make_async_copy(rows[c % NBUF], acc_sh.at[dst_v.at[c]],
                                  ssem[c % NBUF]).wait()

    @pl.when(cid == 0)
    def _core0():
        ring(ysa_hbm)

    @pl.when(cid == 1)
    def _core1():
        ring(ysb_hbm)

    plsc.subcore_barrier()
    pltpu.sync_copy(acc_sh.at[pl.ds(sid * NSUB, NSUB)],
                    p_hbm.at[cid, pl.ds(sid * NSUB, NSUB)])


@functools.cache
def _build_sc_kernels():
    mesh = plsc.VectorSubcoreMesh(
        core_axis_name="c", subcore_axis_name="s",
        num_cores=NC, num_subcores=NS)
    sc_degree = pl.kernel(
        _sc_degree_body,
        out_type=jax.ShapeDtypeStruct((NC * NPAD,), jnp.float32),
        mesh=mesh,
        compiler_params=pltpu.CompilerParams(use_tc_tiling_on_sc=False),
        scratch_types=[
            pltpu.VMEM((CPT_DEG, EC), jnp.int32),
            pltpu.VMEM((EC,), jnp.float32),
            pltpu.VMEM_SHARED((NPAD,), jnp.float32),
            pltpu.SemaphoreType.DMA,
        ],
    )
    sc_agg = pl.kernel(
        _sc_agg_body,
        out_type=jax.ShapeDtypeStruct((NC, N, DH), jnp.float32),
        mesh=mesh,
        compiler_params=pltpu.CompilerParams(use_tc_tiling_on_sc=False),
        scratch_types=(
            [pltpu.VMEM((CPT, EC), jnp.int32),
             pltpu.VMEM((CPT, EC), jnp.int32)]
            + [pltpu.VMEM((EC, DH), jnp.float32) for _ in range(NBUF)]
            + [pltpu.VMEM_SHARED((N, DH), jnp.float32)]
            + [pltpu.SemaphoreType.DMA for _ in range(2 * NBUF)]
        ),
    )
    return sc_degree, sc_agg


def _tc_l1_body(x_ref, w_ref, dg_ref, ysa_ref, ysb_ref):
    dis = lax.rsqrt(dg_ref[...] + 1.0)
    y = jnp.dot(x_ref[...], w_ref[...], preferred_element_type=jnp.float32)
    ys = y * dis
    ysa_ref[...] = ys[:, :DH]
    ysb_ref[...] = ys[:, DH:]


def _tc_l2_body(pa_ref, pb_ref, ysa_ref, ysb_ref, dg_ref, b1_ref,
                w2_ref, y2sa_ref, y2sb_ref):
    dis = lax.rsqrt(dg_ref[...] + 1.0)
    p = jnp.concatenate([pa_ref[0], pb_ref[0]], axis=1)
    ys = jnp.concatenate([ysa_ref[...], ysb_ref[...]], axis=1)
    h = dis * (p + ys) + b1_ref[...]
    h = jnp.maximum(h, 0.0)
    y2s = jnp.dot(h, w2_ref[...], preferred_element_type=jnp.float32) * dis
    y2sa_ref[...] = y2s[:, :DH]
    y2sb_ref[...] = y2s[:, DH:]


def _tc_out_body(pa_ref, pb_ref, y2sa_ref, y2sb_ref, dg_ref, b2_ref,
                 out_ref):
    dis = lax.rsqrt(dg_ref[...] + 1.0)
    p = jnp.concatenate([pa_ref[0], pb_ref[0]], axis=1)
    y2s = jnp.concatenate([y2sa_ref[...], y2sb_ref[...]], axis=1)
    out_ref[...] = dis * (p + y2s) + b2_ref[...]


_row_spec = pl.BlockSpec((TCB, D), lambda i: (i, 0))
_half_spec = pl.BlockSpec((TCB, DH), lambda i: (i, 0))
_pa_spec = pl.BlockSpec((1, TCB, DH), lambda i: (0, i, 0))
_pb_spec = pl.BlockSpec((1, TCB, DH), lambda i: (1, i, 0))
_dg_spec = pl.BlockSpec((TCB, D), lambda i: (i, 0))
_w_spec = pl.BlockSpec((D, D), lambda i: (0, 0))
_b_spec = pl.BlockSpec((1, D), lambda i: (0, 0))

_tc_l1 = pl.pallas_call(
    _tc_l1_body,
    grid=(GRID,),
    in_specs=[_row_spec, _w_spec, _dg_spec],
    out_specs=[_half_spec, _half_spec],
    out_shape=[jax.ShapeDtypeStruct((N, DH), jnp.float32),
               jax.ShapeDtypeStruct((N, DH), jnp.float32)],
)

_tc_l2 = pl.pallas_call(
    _tc_l2_body,
    grid=(GRID,),
    in_specs=[_pa_spec, _pb_spec, _half_spec, _half_spec, _dg_spec,
              _b_spec, _w_spec],
    out_specs=[_half_spec, _half_spec],
    out_shape=[jax.ShapeDtypeStruct((N, DH), jnp.float32),
               jax.ShapeDtypeStruct((N, DH), jnp.float32)],
)

_tc_out = pl.pallas_call(
    _tc_out_body,
    grid=(GRID,),
    in_specs=[_pa_spec, _pb_spec, _half_spec, _half_spec, _dg_spec,
              _b_spec],
    out_specs=_row_spec,
    out_shape=jax.ShapeDtypeStruct((N, D), jnp.float32),
)


def kernel(x, edge_index, W1, b1, W2, b2):
    edges2 = edge_index.reshape(2 * EROWS, EC)
    zeros2 = jnp.zeros((NSUB, DH), jnp.float32)
    zeros1 = jnp.zeros((NPAD,), jnp.float32)
    ones_c = jnp.ones((EC,), jnp.float32)
    b1r = b1.reshape(1, D)
    b2r = b2.reshape(1, D)

    _sc_degree, _sc_agg = _build_sc_kernels()
    deg = _sc_degree(edges2, ones_c, zeros1)
    deg_bc = jnp.broadcast_to(
        (deg[:N] + deg[NPAD:NPAD + N])[:, None], (N, D))

    y1sa, y1sb = _tc_l1(x, W1, deg_bc)
    p1 = _sc_agg(y1sa, y1sb, edges2, zeros2)
    y2sa, y2sb = _tc_l2(p1, p1, y1sa, y1sb, deg_bc, b1r, W2)
    p2 = _sc_agg(y2sa, y2sb, edges2, zeros2)
    return _tc_out(p2, p2, y2sa, y2sb, deg_bc, b2r)
